# baseline (speedup 1.0000x reference)
import jax
import jax.numpy as jnp
from jax import lax
from jax.experimental import pallas as pl

_DEPTH = 1000
_ROWS = 16384
_BD = 200
_BI = 1024


def _one_hot_t_body(idx_ref, out_ref):
    j = pl.program_id(0)
    idx = idx_ref[...]
    rows = lax.broadcasted_iota(jnp.int32, (_BD, _BI), 0) + j * _BD
    out_ref[...] = jnp.where(idx == rows, jnp.float32(1.0), jnp.float32(0.0))


def kernel(inputs):
    idx_t = inputs.astype(jnp.int32).T
    out_t = pl.pallas_call(
        _one_hot_t_body,
        grid=(_DEPTH // _BD, _ROWS // _BI),
        in_specs=[pl.BlockSpec((1, _BI), lambda j, i: (0, i))],
        out_specs=pl.BlockSpec((_BD, _BI), lambda j, i: (j, i)),
        out_shape=jax.ShapeDtypeStruct((_DEPTH, _ROWS), jnp.float32),
    )(idx_t)
    return out_t.T


# speedup vs baseline: 2.3414x; 2.3414x over previous
import jax
import jax.numpy as jnp
from jax import lax
from jax.experimental import pallas as pl

_DEPTH = 1000
_ROWS = 16384
_BI = 1024


def _one_hot_t_body(idx_ref, out_ref):
    idx = idx_ref[...]
    rows = lax.broadcasted_iota(jnp.int32, (_DEPTH, _BI), 0)
    out_ref[...] = jnp.where(idx == rows, jnp.float32(1.0), jnp.float32(0.0))


def kernel(inputs):
    idx_t = inputs.astype(jnp.int32).T
    out_t = pl.pallas_call(
        _one_hot_t_body,
        grid=(_ROWS // _BI,),
        in_specs=[pl.BlockSpec((1, _BI), lambda i: (0, i))],
        out_specs=pl.BlockSpec((_DEPTH, _BI), lambda i: (0, i)),
        out_shape=jax.ShapeDtypeStruct((_DEPTH, _ROWS), jnp.float32),
    )(idx_t)
    return out_t.T
